# Initial kernel scaffold; baseline (speedup 1.0000x reference)
import functools

import jax
import jax.numpy as jnp
from jax import lax
from jax.experimental import pallas as pl
from jax.experimental.pallas import tpu as pltpu
from jax.experimental.pallas import tpu_sc as plsc

N = 10000
NP = 10016
E = 320000
H = 16
OUT = 40
NC = 2
NS = 16
NW = NC * NS
CH = 128
NCH = 79
EP = NW * NCH * CH
RPT = NP // NS
PAD_NODE = N + 8
NI = 1000
NI_P = 1024

_mesh = plsc.VectorSubcoreMesh(core_axis_name="c", subcore_axis_name="s")


@functools.partial(
    pl.kernel,
    out_type=jax.ShapeDtypeStruct((NC * NP, H), jnp.float32),
    mesh=_mesh,
    scratch_types=[
        pltpu.VMEM((NCH, CH), jnp.int32),
        pltpu.VMEM((CH, H), jnp.float32),
        pltpu.VMEM_SHARED((NP, H), jnp.float32),
    ],
)
def _deg_call(dstI, ones, zeros, outP, dst_v, ones_v, acc):
    cid = lax.axis_index("c")
    sid = lax.axis_index("s")
    wid = sid * NC + cid
    pltpu.sync_copy(dstI.at[wid], dst_v)
    pltpu.sync_copy(ones, ones_v)
    pltpu.sync_copy(zeros.at[pl.ds(sid * RPT, RPT)], acc.at[pl.ds(sid * RPT, RPT)])
    plsc.subcore_barrier()

    def chunk(j, carry):
        pltpu.sync_copy(ones_v, acc.at[dst_v.at[j]], add=True)
        return carry

    lax.fori_loop(0, NCH, chunk, 0)
    plsc.subcore_barrier()
    pltpu.sync_copy(acc.at[pl.ds(sid * RPT, RPT)],
                    outP.at[pl.ds(cid * NP + sid * RPT, RPT)])


@functools.partial(
    pl.kernel,
    out_type=jax.ShapeDtypeStruct((NC * NP, H), jnp.float32),
    mesh=_mesh,
    scratch_types=[
        pltpu.VMEM((NCH, CH), jnp.int32),
        pltpu.VMEM((NCH, CH), jnp.int32),
        pltpu.VMEM((CH, H), jnp.float32),
        pltpu.VMEM_SHARED((NP, H), jnp.float32),
        pltpu.SemaphoreType.DMA,
    ],
)
def _prop_call(g, srcI, dstI, zeros, outP, src_v, dst_v, rows_v, acc, sem):
    cid = lax.axis_index("c")
    sid = lax.axis_index("s")
    wid = sid * NC + cid
    pltpu.sync_copy(srcI.at[wid], src_v)
    pltpu.sync_copy(dstI.at[wid], dst_v)
    pltpu.sync_copy(zeros.at[pl.ds(sid * RPT, RPT)], acc.at[pl.ds(sid * RPT, RPT)])
    plsc.subcore_barrier()

    def chunk(j, carry):
        pltpu.async_copy(g.at[src_v.at[j]], rows_v, sem).wait()
        pltpu.sync_copy(rows_v, acc.at[dst_v.at[j]], add=True)
        return carry

    lax.fori_loop(0, NCH, chunk, 0)
    plsc.subcore_barrier()
    pltpu.sync_copy(acc.at[pl.ds(sid * RPT, RPT)],
                    outP.at[pl.ds(cid * NP + sid * RPT, RPT)])


@functools.partial(
    pl.kernel,
    out_type=(jax.ShapeDtypeStruct((NC * NI_P, H), jnp.float32),
              jax.ShapeDtypeStruct((NI_P, H), jnp.float32)),
    mesh=_mesh,
    scratch_types=[
        pltpu.VMEM((NCH, CH), jnp.int32),
        pltpu.VMEM((NCH, CH), jnp.int32),
        pltpu.VMEM((CH, H), jnp.float32),
        pltpu.VMEM((CH,), jnp.int32),
        pltpu.VMEM((CH, H), jnp.float32),
        pltpu.VMEM_SHARED((NP, H), jnp.float32),
        pltpu.SemaphoreType.DMA,
    ],
)
def _propg_call(g, srcI, dstI, niI, dinv, G, Dni,
                src_v, dst_v, rows_v, ni_v, grows_v, acc, sem):
    cid = lax.axis_index("c")
    sid = lax.axis_index("s")
    wid = sid * NC + cid
    pltpu.sync_copy(srcI.at[wid], src_v)
    pltpu.sync_copy(dstI.at[wid], dst_v)
    pltpu.sync_copy(dinv.at[pl.ds(0, RPT)], acc.at[pl.ds(sid * RPT, RPT)]) if False else None
    pltpu.sync_copy(g.at[pl.ds(sid * RPT, RPT)], acc.at[pl.ds(sid * RPT, RPT)]) if False else None
    plsc.subcore_barrier()


def kernel(x, edge_index, node_index, W1, W2):
    raise NotImplementedError


# speedup vs baseline: 29.9431x; 29.9431x over previous
import functools

import jax
import jax.numpy as jnp
from jax import lax
from jax.experimental import pallas as pl
from jax.experimental.pallas import tpu as pltpu
from jax.experimental.pallas import tpu_sc as plsc

N = 10000
NP = 10112
E = 320000
H = 16
OUT = 40
NC = 2
NS = 16
NW = NC * NS
CH = 128
NCH = 79
EP = NW * NCH * CH
RPT = NP // NS
PAD_NODE = N + 8
NI = 1000
NI_P = 1024

def _deg_body(dstI, ones, zeros, outP, dst_v, ones_v, acc):
    cid = lax.axis_index("c")
    sid = lax.axis_index("s")
    wid = sid * NC + cid
    pltpu.sync_copy(dstI.at[wid], dst_v)
    pltpu.sync_copy(ones, ones_v)
    pltpu.sync_copy(zeros.at[pl.ds(sid * RPT, RPT)], acc.at[pl.ds(sid * RPT, RPT)])
    plsc.subcore_barrier()

    def chunk(j, carry):
        pltpu.sync_copy(ones_v, acc.at[dst_v.at[j]], add=True)
        return carry

    lax.fori_loop(0, NCH, chunk, 0)
    plsc.subcore_barrier()
    pltpu.sync_copy(acc.at[pl.ds(sid * RPT, RPT)],
                    outP.at[pl.ds(cid * NP + sid * RPT, RPT)])


def _prop_body(g, srcI, dstI, zeros, outP, src_v, dst_v, rows_v, acc, sem):
    cid = lax.axis_index("c")
    sid = lax.axis_index("s")
    wid = sid * NC + cid
    pltpu.sync_copy(srcI.at[wid], src_v)
    pltpu.sync_copy(dstI.at[wid], dst_v)
    pltpu.sync_copy(zeros.at[pl.ds(sid * RPT, RPT)], acc.at[pl.ds(sid * RPT, RPT)])
    plsc.subcore_barrier()

    def chunk(j, carry):
        pltpu.async_copy(g.at[src_v.at[j]], rows_v, sem).wait()
        pltpu.sync_copy(rows_v, acc.at[dst_v.at[j]], add=True)
        return carry

    lax.fori_loop(0, NCH, chunk, 0)
    plsc.subcore_barrier()
    pltpu.sync_copy(acc.at[pl.ds(sid * RPT, RPT)],
                    outP.at[pl.ds(cid * NP + sid * RPT, RPT)])


def _propg_body(g, srcI, dstI, niI, dinv, zeros, G, Dni,
                src_v, dst_v, rows_v, ni_v, grows_v, acc, sem):
    cid = lax.axis_index("c")
    sid = lax.axis_index("s")
    wid = sid * NC + cid
    pltpu.sync_copy(srcI.at[wid], src_v)
    pltpu.sync_copy(dstI.at[wid], dst_v)
    pltpu.sync_copy(zeros.at[pl.ds(sid * RPT, RPT)], acc.at[pl.ds(sid * RPT, RPT)])
    plsc.subcore_barrier()

    def chunk(j, carry):
        pltpu.async_copy(g.at[src_v.at[j]], rows_v, sem).wait()
        pltpu.sync_copy(rows_v, acc.at[dst_v.at[j]], add=True)
        return carry

    lax.fori_loop(0, NCH, chunk, 0)
    plsc.subcore_barrier()

    @pl.when(sid < 8)
    def _():
        pltpu.sync_copy(niI.at[pl.ds(sid * CH, CH)], ni_v)
        pltpu.async_copy(acc.at[ni_v], grows_v, sem).wait()
        pltpu.sync_copy(grows_v, G.at[pl.ds(cid * NI_P + sid * CH, CH)])

    @pl.when((sid >= 8) & (cid == 0))
    def _():
        pltpu.sync_copy(niI.at[pl.ds((sid - 8) * CH, CH)], ni_v)
        pltpu.async_copy(dinv.at[ni_v], grows_v, sem).wait()
        pltpu.sync_copy(grows_v, Dni.at[pl.ds((sid - 8) * CH, CH)])


@functools.lru_cache(maxsize=None)
def _sc_kernels():
    mesh = plsc.VectorSubcoreMesh(core_axis_name="c", subcore_axis_name="s",
                                  num_cores=NC, num_subcores=NS)
    params = pltpu.CompilerParams(use_tc_tiling_on_sc=False)
    deg = pl.kernel(
        _deg_body,
        out_type=jax.ShapeDtypeStruct((NC * NP, H), jnp.float32),
        mesh=mesh,
        compiler_params=params,
        scratch_types=[
            pltpu.VMEM((NCH, CH), jnp.int32),
            pltpu.VMEM((CH, H), jnp.float32),
            pltpu.VMEM_SHARED((NP, H), jnp.float32),
        ],
    )
    prop = pl.kernel(
        _prop_body,
        out_type=jax.ShapeDtypeStruct((NC * NP, H), jnp.float32),
        mesh=mesh,
        compiler_params=params,
        scratch_types=[
            pltpu.VMEM((NCH, CH), jnp.int32),
            pltpu.VMEM((NCH, CH), jnp.int32),
            pltpu.VMEM((CH, H), jnp.float32),
            pltpu.VMEM_SHARED((NP, H), jnp.float32),
            pltpu.SemaphoreType.DMA,
        ],
    )
    propg = pl.kernel(
        _propg_body,
        out_type=(jax.ShapeDtypeStruct((NC * NI_P, H), jnp.float32),
                  jax.ShapeDtypeStruct((NI_P, H), jnp.float32)),
        mesh=mesh,
        compiler_params=params,
        scratch_types=[
            pltpu.VMEM((NCH, CH), jnp.int32),
            pltpu.VMEM((NCH, CH), jnp.int32),
            pltpu.VMEM((CH, H), jnp.float32),
            pltpu.VMEM((CH,), jnp.int32),
            pltpu.VMEM((CH, H), jnp.float32),
            pltpu.VMEM_SHARED((NP, H), jnp.float32),
            pltpu.SemaphoreType.DMA,
        ],
    )
    return deg, prop, propg


def _fuse1_body(x_ref, w1_ref, degp_ref, dinv_ref, g0_ref):
    deg = jnp.maximum(degp_ref[0] + degp_ref[1], 1.0)
    dinv = lax.rsqrt(deg)
    dinv_ref[...] = dinv
    g0_ref[...] = jnp.dot(x_ref[...], w1_ref[...],
                          preferred_element_type=jnp.float32) * dinv


_fuse1 = pl.pallas_call(
    _fuse1_body,
    out_shape=(jax.ShapeDtypeStruct((NP, H), jnp.float32),
               jax.ShapeDtypeStruct((NP, H), jnp.float32)),
)


def _fuse2_body(pp_ref, dinv_ref, g1_ref):
    s = (pp_ref[0] + pp_ref[1]) * dinv_ref[...]
    g1_ref[...] = jnp.maximum(s, 0.0) * dinv_ref[...]


_fuse2 = pl.pallas_call(
    _fuse2_body,
    out_shape=jax.ShapeDtypeStruct((NP, H), jnp.float32),
)


def _fuse3_body(gg_ref, dni_ref, w2_ref, out_ref):
    rows = (gg_ref[0] + gg_ref[1]) * dni_ref[...]
    out_ref[...] = jnp.dot(rows, w2_ref[...],
                           preferred_element_type=jnp.float32)


_fuse3 = pl.pallas_call(
    _fuse3_body,
    out_shape=jax.ShapeDtypeStruct((NI_P, OUT), jnp.float32),
)


def kernel(x, edge_index, node_index, W1, W2):
    src = edge_index[0].astype(jnp.int32)
    dst = edge_index[1].astype(jnp.int32)
    epad = jnp.full((EP - E,), PAD_NODE, jnp.int32)
    srcI = jnp.concatenate([src, epad]).reshape(NW, NCH, CH)
    dstI = jnp.concatenate([dst, epad]).reshape(NW, NCH, CH)
    ni = jnp.concatenate([node_index.astype(jnp.int32),
                          jnp.zeros((NI_P - NI,), jnp.int32)])
    xp = jnp.pad(x, ((0, NP - N), (0, 0)))
    zeros = jnp.zeros((NP, H), jnp.float32)
    ones = jnp.ones((CH, H), jnp.float32)

    _deg_call, _prop_call, _propg_call = _sc_kernels()
    degp = _deg_call(dstI, ones, zeros).reshape(NC, NP, H)
    dinv, g0 = _fuse1(xp, W1, degp)
    p1 = _prop_call(g0, srcI, dstI, zeros).reshape(NC, NP, H)
    g1 = _fuse2(p1, dinv)
    G, Dni = _propg_call(g1, srcI, dstI, ni, dinv, zeros)
    out = _fuse3(G.reshape(NC, NI_P, H), Dni, W2)
    return out[:NI]


# speedup vs baseline: 49.6376x; 1.6577x over previous
import functools

import jax
import jax.numpy as jnp
from jax import lax
from jax.experimental import pallas as pl
from jax.experimental.pallas import tpu as pltpu
from jax.experimental.pallas import tpu_sc as plsc

N = 10000
NP = 10112
E = 320000
H = 16
OUT = 40
NC = 2
NS = 16
NW = NC * NS
CH = 128
NCH = 80
NBUF = 4
NGRP = NCH // NBUF
EP = NW * NCH * CH
RPT = NP // NS
PAD_NODE = N + 8
NI = 1000
NI_P = 1024

def _deg_body(dstI, ones, zeros, outP, dst_v, ones_v, acc, sem):
    cid = lax.axis_index("c")
    sid = lax.axis_index("s")
    wid = sid * NC + cid
    pltpu.sync_copy(dstI.at[wid], dst_v)
    pltpu.sync_copy(ones, ones_v)
    pltpu.sync_copy(zeros.at[pl.ds(sid * RPT, RPT)], acc.at[pl.ds(sid * RPT, RPT)])
    plsc.subcore_barrier()

    K = 8

    def group(g_i, carry):
        for b in range(K):
            pltpu.async_copy(ones_v, acc.at[dst_v.at[g_i * K + b]], sem,
                             add=True)
        for b in range(K):
            pltpu.make_async_copy(ones_v, acc.at[dst_v.at[g_i * K + b]],
                                  sem).wait()
        return carry

    lax.fori_loop(0, NCH // K, group, 0)
    plsc.subcore_barrier()
    pltpu.sync_copy(acc.at[pl.ds(sid * RPT, RPT)],
                    outP.at[pl.ds(cid * NP + sid * RPT, RPT)])


def _prop_body(g, srcI, dstI, zeros, outP, src_v, dst_v, rows_v, acc, sem):
    cid = lax.axis_index("c")
    sid = lax.axis_index("s")
    wid = sid * NC + cid
    pltpu.sync_copy(srcI.at[wid], src_v)
    pltpu.sync_copy(dstI.at[wid], dst_v)
    pltpu.sync_copy(zeros.at[pl.ds(sid * RPT, RPT)], acc.at[pl.ds(sid * RPT, RPT)])
    plsc.subcore_barrier()

    for b in range(NBUF):
        pltpu.async_copy(g.at[src_v.at[b]], rows_v.at[b], sem)

    def group(g_i, carry):
        for b in range(NBUF):
            j = g_i * NBUF + b
            pltpu.make_async_copy(g.at[src_v.at[j]], rows_v.at[b], sem).wait()
            pltpu.sync_copy(rows_v.at[b], acc.at[dst_v.at[j]], add=True)
            pltpu.async_copy(g.at[src_v.at[j + NBUF]], rows_v.at[b], sem)
        return carry

    lax.fori_loop(0, NGRP - 1, group, 0)
    for b in range(NBUF):
        j = (NGRP - 1) * NBUF + b
        pltpu.make_async_copy(g.at[src_v.at[j]], rows_v.at[b], sem).wait()
        pltpu.sync_copy(rows_v.at[b], acc.at[dst_v.at[j]], add=True)
    plsc.subcore_barrier()
    pltpu.sync_copy(acc.at[pl.ds(sid * RPT, RPT)],
                    outP.at[pl.ds(cid * NP + sid * RPT, RPT)])


def _propg_body(g, srcI, dstI, niI, dinv, zeros, G, Dni,
                src_v, dst_v, rows_v, ni_v, grows_v, acc, sem):
    cid = lax.axis_index("c")
    sid = lax.axis_index("s")
    wid = sid * NC + cid
    pltpu.sync_copy(srcI.at[wid], src_v)
    pltpu.sync_copy(dstI.at[wid], dst_v)
    pltpu.sync_copy(zeros.at[pl.ds(sid * RPT, RPT)], acc.at[pl.ds(sid * RPT, RPT)])
    plsc.subcore_barrier()

    for b in range(NBUF):
        pltpu.async_copy(g.at[src_v.at[b]], rows_v.at[b], sem)

    def group(g_i, carry):
        for b in range(NBUF):
            j = g_i * NBUF + b
            pltpu.make_async_copy(g.at[src_v.at[j]], rows_v.at[b], sem).wait()
            pltpu.sync_copy(rows_v.at[b], acc.at[dst_v.at[j]], add=True)
            pltpu.async_copy(g.at[src_v.at[j + NBUF]], rows_v.at[b], sem)
        return carry

    lax.fori_loop(0, NGRP - 1, group, 0)
    for b in range(NBUF):
        j = (NGRP - 1) * NBUF + b
        pltpu.make_async_copy(g.at[src_v.at[j]], rows_v.at[b], sem).wait()
        pltpu.sync_copy(rows_v.at[b], acc.at[dst_v.at[j]], add=True)
    plsc.subcore_barrier()

    @pl.when(sid < 8)
    def _():
        pltpu.sync_copy(niI.at[pl.ds(sid * CH, CH)], ni_v)
        pltpu.async_copy(acc.at[ni_v], grows_v, sem).wait()
        pltpu.sync_copy(grows_v, G.at[pl.ds(cid * NI_P + sid * CH, CH)])

    @pl.when((sid >= 8) & (cid == 0))
    def _():
        pltpu.sync_copy(niI.at[pl.ds((sid - 8) * CH, CH)], ni_v)
        pltpu.async_copy(dinv.at[ni_v], grows_v, sem).wait()
        pltpu.sync_copy(grows_v, Dni.at[pl.ds((sid - 8) * CH, CH)])


@functools.lru_cache(maxsize=None)
def _sc_kernels():
    mesh = plsc.VectorSubcoreMesh(core_axis_name="c", subcore_axis_name="s",
                                  num_cores=NC, num_subcores=NS)
    params = pltpu.CompilerParams(use_tc_tiling_on_sc=False)
    deg = pl.kernel(
        _deg_body,
        out_type=jax.ShapeDtypeStruct((NC * NP, H), jnp.float32),
        mesh=mesh,
        compiler_params=params,
        scratch_types=[
            pltpu.VMEM((NCH, CH), jnp.int32),
            pltpu.VMEM((CH, H), jnp.float32),
            pltpu.VMEM_SHARED((NP, H), jnp.float32),
            pltpu.SemaphoreType.DMA,
        ],
    )
    prop = pl.kernel(
        _prop_body,
        out_type=jax.ShapeDtypeStruct((NC * NP, H), jnp.float32),
        mesh=mesh,
        compiler_params=params,
        scratch_types=[
            pltpu.VMEM((NCH, CH), jnp.int32),
            pltpu.VMEM((NCH, CH), jnp.int32),
            pltpu.VMEM((NBUF, CH, H), jnp.float32),
            pltpu.VMEM_SHARED((NP, H), jnp.float32),
            pltpu.SemaphoreType.DMA,
        ],
    )
    propg = pl.kernel(
        _propg_body,
        out_type=(jax.ShapeDtypeStruct((NC * NI_P, H), jnp.float32),
                  jax.ShapeDtypeStruct((NI_P, H), jnp.float32)),
        mesh=mesh,
        compiler_params=params,
        scratch_types=[
            pltpu.VMEM((NCH, CH), jnp.int32),
            pltpu.VMEM((NCH, CH), jnp.int32),
            pltpu.VMEM((NBUF, CH, H), jnp.float32),
            pltpu.VMEM((CH,), jnp.int32),
            pltpu.VMEM((CH, H), jnp.float32),
            pltpu.VMEM_SHARED((NP, H), jnp.float32),
            pltpu.SemaphoreType.DMA,
        ],
    )
    return deg, prop, propg


def _fuse1_body(x_ref, w1_ref, degp_ref, dinv_ref, g0_ref):
    deg = jnp.maximum(degp_ref[0] + degp_ref[1], 1.0)
    dinv = lax.rsqrt(deg)
    dinv_ref[...] = dinv
    g0_ref[...] = jnp.dot(x_ref[...], w1_ref[...],
                          preferred_element_type=jnp.float32) * dinv


_fuse1 = pl.pallas_call(
    _fuse1_body,
    out_shape=(jax.ShapeDtypeStruct((NP, H), jnp.float32),
               jax.ShapeDtypeStruct((NP, H), jnp.float32)),
)


def _fuse2_body(pp_ref, dinv_ref, g1_ref):
    s = (pp_ref[0] + pp_ref[1]) * dinv_ref[...]
    g1_ref[...] = jnp.maximum(s, 0.0) * dinv_ref[...]


_fuse2 = pl.pallas_call(
    _fuse2_body,
    out_shape=jax.ShapeDtypeStruct((NP, H), jnp.float32),
)


def _fuse3_body(gg_ref, dni_ref, w2_ref, out_ref):
    rows = (gg_ref[0] + gg_ref[1]) * dni_ref[...]
    out_ref[...] = jnp.dot(rows, w2_ref[...],
                           preferred_element_type=jnp.float32)


_fuse3 = pl.pallas_call(
    _fuse3_body,
    out_shape=jax.ShapeDtypeStruct((NI_P, OUT), jnp.float32),
)


def kernel(x, edge_index, node_index, W1, W2):
    src = edge_index[0].astype(jnp.int32)
    dst = edge_index[1].astype(jnp.int32)
    epad = N + jnp.arange(EP - E, dtype=jnp.int32) % (NP - N)
    srcI = jnp.concatenate([src, epad]).reshape(NW, NCH, CH)
    dstI = jnp.concatenate([dst, epad]).reshape(NW, NCH, CH)
    ni = jnp.concatenate([node_index.astype(jnp.int32),
                          jnp.zeros((NI_P - NI,), jnp.int32)])
    xp = jnp.pad(x, ((0, NP - N), (0, 0)))
    zeros = jnp.zeros((NP, H), jnp.float32)
    ones = jnp.ones((CH, H), jnp.float32)

    _deg_call, _prop_call, _propg_call = _sc_kernels()
    degp = _deg_call(dstI, ones, zeros).reshape(NC, NP, H)
    dinv, g0 = _fuse1(xp, W1, degp)
    p1 = _prop_call(g0, srcI, dstI, zeros).reshape(NC, NP, H)
    g1 = _fuse2(p1, dinv)
    G, Dni = _propg_call(g1, srcI, dstI, ni, dinv, zeros)
    out = _fuse3(G.reshape(NC, NI_P, H), Dni, W2)
    return out[:NI]


# speedup vs baseline: 69.7211x; 1.4046x over previous
import functools

import jax
import jax.numpy as jnp
from jax import lax
from jax.experimental import pallas as pl
from jax.experimental.pallas import tpu as pltpu
from jax.experimental.pallas import tpu_sc as plsc

N = 10000
NP = 10112
E = 320000
H = 16
OUT = 40
NC = 2
NS = 16
NW = NC * NS
CH = 128
NCT = E // CH
NCW = 78
NT = NCT - NW * NCW
RPT = NP // NS
NBUF = 6
D = 3
NGRP = NCW // NBUF
NI = 1000
NI_P = 1024


def _worker_ids():
    cid = lax.axis_index("c")
    sid = lax.axis_index("s")
    return cid, sid, sid * NC + cid


def _stage_chunks(EI, row, idx_v, wid):
    pltpu.sync_copy(EI.at[row, pl.ds(wid * NCW, NCW)], idx_v.at[pl.ds(0, NCW)])

    @pl.when(wid < NT)
    def _():
        pltpu.sync_copy(EI.at[row, pl.ds(NW * NCW + wid, 1)],
                        idx_v.at[pl.ds(NCW, 1)])


def _deg_body(EI, ones, zeros1, outP, dst_v, ones_v, acc, sem):
    cid, sid, wid = _worker_ids()
    _stage_chunks(EI, 1, dst_v, wid)
    pltpu.sync_copy(ones, ones_v)
    pltpu.sync_copy(zeros1.at[pl.ds(sid * RPT, RPT)],
                    acc.at[pl.ds(sid * RPT, RPT)])
    plsc.subcore_barrier()

    K = 6

    def group(gi, c):
        for b in range(K):
            pltpu.async_copy(ones_v, acc.at[dst_v.at[gi * K + b]], sem,
                             add=True)
        for b in range(K):
            pltpu.make_async_copy(ones_v, acc.at[dst_v.at[gi * K + b]],
                                  sem).wait()
        return c

    lax.fori_loop(0, NCW // K, group, 0)

    @pl.when(wid < NT)
    def _():
        pltpu.sync_copy(ones_v, acc.at[dst_v.at[NCW]], add=True)

    plsc.subcore_barrier()
    pltpu.sync_copy(acc.at[pl.ds(sid * RPT, RPT)],
                    outP.at[pl.ds(cid * NP + sid * RPT, RPT)])


def _prop_loop(SRC, src_v, dst_v, rows_v, acc, sem_g, sem_s, wid):
    for d in range(D):
        pltpu.async_copy(SRC.at[src_v.at[d]], rows_v.at[d], sem_g)

    def group(gi, c):
        for b in range(NBUF):
            j = gi * NBUF + b
            pltpu.make_async_copy(SRC.at[src_v.at[j]], rows_v.at[b],
                                  sem_g).wait()
            pltpu.async_copy(rows_v.at[b], acc.at[dst_v.at[j]], sem_s,
                             add=True)

            @pl.when(j >= NBUF - D)
            def _():
                pltpu.make_async_copy(rows_v.at[b], acc.at[dst_v.at[b]],
                                      sem_s).wait()

            @pl.when(j + D < NCW)
            def _():
                pltpu.async_copy(SRC.at[src_v.at[j + D]],
                                 rows_v.at[(b + D) % NBUF], sem_g)

        return c

    lax.fori_loop(0, NGRP, group, 0)

    @pl.when(wid < NT)
    def _():
        pltpu.async_copy(SRC.at[src_v.at[NCW]], rows_v.at[0], sem_g)
        pltpu.make_async_copy(SRC.at[src_v.at[NCW]], rows_v.at[0],
                              sem_g).wait()
        pltpu.async_copy(rows_v.at[0], acc.at[dst_v.at[NCW]], sem_s,
                         add=True)
        pltpu.make_async_copy(rows_v.at[0], acc.at[dst_v.at[NCW]],
                              sem_s).wait()

    for k in range(NBUF - D):
        pltpu.make_async_copy(rows_v.at[k], acc.at[dst_v.at[k]], sem_s).wait()


def _prop1_body(g, EI, zeros16, outP, src_v, dst_v, rows_v, acc, sem_g, sem_s):
    cid, sid, wid = _worker_ids()
    _stage_chunks(EI, 0, src_v, wid)
    _stage_chunks(EI, 1, dst_v, wid)
    pltpu.sync_copy(zeros16.at[pl.ds(sid * RPT, RPT)],
                    acc.at[pl.ds(sid * RPT, RPT)])
    plsc.subcore_barrier()
    _prop_loop(g, src_v, dst_v, rows_v, acc, sem_g, sem_s, wid)
    plsc.subcore_barrier()
    pltpu.sync_copy(acc.at[pl.ds(sid * RPT, RPT)],
                    outP.at[pl.ds(cid * NP + sid * RPT, RPT)])


def _prop2g_body(pP, EI, niI, dinv, zeros16, G, Dni,
                 src_v, dst_v, rows_v, pv0, pv1, dv, gv, ni_v, grows_v,
                 g1s, acc, sem_g, sem_s):
    cid, sid, wid = _worker_ids()
    _stage_chunks(EI, 0, src_v, wid)
    _stage_chunks(EI, 1, dst_v, wid)
    pltpu.sync_copy(zeros16.at[pl.ds(sid * RPT, RPT)],
                    acc.at[pl.ds(sid * RPT, RPT)])

    r0 = sid * RPT
    pltpu.sync_copy(pP.at[pl.ds(r0, RPT)], pv0)
    pltpu.sync_copy(pP.at[pl.ds(NP + r0, RPT)], pv1)
    pltpu.sync_copy(dinv.at[pl.ds(r0, RPT)], dv)

    def row(i, c):
        t = (pv0[i] + pv1[i]) * dv[i]
        gv[i] = jnp.maximum(t, 0.0) * dv[i]
        return c

    lax.fori_loop(0, RPT, row, 0)
    pltpu.sync_copy(gv, g1s.at[pl.ds(r0, RPT)])
    plsc.subcore_barrier()

    _prop_loop(g1s, src_v, dst_v, rows_v, acc, sem_g, sem_s, wid)
    plsc.subcore_barrier()

    @pl.when(sid < 8)
    def _():
        pltpu.sync_copy(niI.at[pl.ds(sid * CH, CH)], ni_v)
        pltpu.async_copy(acc.at[ni_v], grows_v, sem_g).wait()
        pltpu.sync_copy(grows_v, G.at[pl.ds(cid * NI_P + sid * CH, CH)])

    @pl.when((sid >= 8) & (cid == 0))
    def _():
        pltpu.sync_copy(niI.at[pl.ds((sid - 8) * CH, CH)], ni_v)
        pltpu.async_copy(dinv.at[ni_v], grows_v, sem_g).wait()
        pltpu.sync_copy(grows_v, Dni.at[pl.ds((sid - 8) * CH, CH)])


@functools.lru_cache(maxsize=None)
def _sc_kernels():
    mesh = plsc.VectorSubcoreMesh(core_axis_name="c", subcore_axis_name="s",
                                  num_cores=NC, num_subcores=NS)
    params = pltpu.CompilerParams(use_tc_tiling_on_sc=False)
    deg = pl.kernel(
        _deg_body,
        out_type=jax.ShapeDtypeStruct((NC * NP,), jnp.float32),
        mesh=mesh,
        compiler_params=params,
        scratch_types=[
            pltpu.VMEM((NCW + 1, CH), jnp.int32),
            pltpu.VMEM((CH,), jnp.float32),
            pltpu.VMEM_SHARED((NP,), jnp.float32),
            pltpu.SemaphoreType.DMA,
        ],
    )
    prop1 = pl.kernel(
        _prop1_body,
        out_type=jax.ShapeDtypeStruct((NC * NP, H), jnp.float32),
        mesh=mesh,
        compiler_params=params,
        scratch_types=[
            pltpu.VMEM((NCW + 1, CH), jnp.int32),
            pltpu.VMEM((NCW + 1, CH), jnp.int32),
            pltpu.VMEM((NBUF, CH, H), jnp.float32),
            pltpu.VMEM_SHARED((NP, H), jnp.float32),
            pltpu.SemaphoreType.DMA,
            pltpu.SemaphoreType.DMA,
        ],
    )
    prop2g = pl.kernel(
        _prop2g_body,
        out_type=(jax.ShapeDtypeStruct((NC * NI_P, H), jnp.float32),
                  jax.ShapeDtypeStruct((NI_P, H), jnp.float32)),
        mesh=mesh,
        compiler_params=params,
        scratch_types=[
            pltpu.VMEM((NCW + 1, CH), jnp.int32),
            pltpu.VMEM((NCW + 1, CH), jnp.int32),
            pltpu.VMEM((NBUF, CH, H), jnp.float32),
            pltpu.VMEM((RPT, H), jnp.float32),
            pltpu.VMEM((RPT, H), jnp.float32),
            pltpu.VMEM((RPT, H), jnp.float32),
            pltpu.VMEM((RPT, H), jnp.float32),
            pltpu.VMEM((CH,), jnp.int32),
            pltpu.VMEM((CH, H), jnp.float32),
            pltpu.VMEM_SHARED((NP, H), jnp.float32),
            pltpu.VMEM_SHARED((NP, H), jnp.float32),
            pltpu.SemaphoreType.DMA,
            pltpu.SemaphoreType.DMA,
        ],
    )
    return deg, prop1, prop2g


def _fuse1_body(x_ref, w1_ref, degp_ref, dinv_ref, g0_ref):
    deg = jnp.maximum(degp_ref[:NP] + degp_ref[NP:], 1.0)
    dinv = lax.rsqrt(deg)
    dinv16 = jnp.broadcast_to(dinv[:, None], (NP, H))
    dinv_ref[...] = dinv16
    xw = jnp.dot(x_ref[...], w1_ref[...], preferred_element_type=jnp.float32)
    xwp = jnp.concatenate([xw, jnp.zeros((NP - N, H), jnp.float32)], axis=0)
    g0_ref[...] = xwp * dinv16


_fuse1 = pl.pallas_call(
    _fuse1_body,
    out_shape=(jax.ShapeDtypeStruct((NP, H), jnp.float32),
               jax.ShapeDtypeStruct((NP, H), jnp.float32)),
)


def _fuse3_body(G_ref, dni_ref, w2_ref, out_ref):
    rows = (G_ref[:NI_P] + G_ref[NI_P:]) * dni_ref[...]
    out_ref[...] = jnp.dot(rows, w2_ref[...],
                           preferred_element_type=jnp.float32)


_fuse3 = pl.pallas_call(
    _fuse3_body,
    out_shape=jax.ShapeDtypeStruct((NI_P, OUT), jnp.float32),
)


def kernel(x, edge_index, node_index, W1, W2):
    deg_k, prop1_k, prop2g_k = _sc_kernels()
    EI = edge_index.astype(jnp.int32).reshape(2, NCT, CH)
    niP = jnp.concatenate([node_index.astype(jnp.int32),
                           jnp.zeros((NI_P - NI,), jnp.int32)])
    zeros1 = jnp.zeros((NP,), jnp.float32)
    zeros16 = jnp.zeros((NP, H), jnp.float32)
    ones1 = jnp.ones((CH,), jnp.float32)

    degp = deg_k(EI, ones1, zeros1)
    dinv16, g0 = _fuse1(x, W1, degp)
    pP = prop1_k(g0, EI, zeros16)
    G, Dni = prop2g_k(pP, EI, niP, dinv16, zeros16)
    out = _fuse3(G, Dni, W2)
    return out[:NI]


# speedup vs baseline: 83.0089x; 1.1906x over previous
import functools

import jax
import jax.numpy as jnp
from jax import lax
from jax.experimental import pallas as pl
from jax.experimental.pallas import tpu as pltpu
from jax.experimental.pallas import tpu_sc as plsc

N = 10000
NP = 10112
E = 320000
H = 16
OUT = 40
NC = 2
NS = 16
NW = NC * NS
CH = 128
NCT = E // CH
NCW = 78
NT = NCT - NW * NCW
RPT = NP // NS
NBUF = 6
D = 3
NGRP = NCW // NBUF
NI = 1000
NI_P = 1024


def _worker_ids():
    cid = lax.axis_index("c")
    sid = lax.axis_index("s")
    return cid, sid, sid * NC + cid


EW = NCW * CH


def _stage_chunks(EI, row, idx_v, wid):
    pltpu.sync_copy(EI.at[row, pl.ds(wid * EW, EW)], idx_v.at[pl.ds(0, EW)])

    @pl.when(wid < NT)
    def _():
        pltpu.sync_copy(EI.at[row, pl.ds(NW * EW + wid * CH, CH)],
                        idx_v.at[pl.ds(EW, CH)])


def _deg_body(EI, ones, zeros1, outP, dst_v, ones_v, acc, sem):
    cid, sid, wid = _worker_ids()
    _stage_chunks(EI, 1, dst_v, wid)
    pltpu.sync_copy(ones, ones_v)
    pltpu.sync_copy(zeros1.at[pl.ds(sid * RPT, RPT)],
                    acc.at[pl.ds(sid * RPT, RPT)])
    plsc.subcore_barrier()

    K = 6

    def group(gi, c):
        for b in range(K):
            pltpu.async_copy(ones_v, acc.at[_ck(dst_v, gi * K + b)], sem,
                             add=True)
        for b in range(K):
            pltpu.make_async_copy(ones_v, acc.at[_ck(dst_v, gi * K + b)],
                                  sem).wait()
        return c

    lax.fori_loop(0, NCW // K, group, 0)

    @pl.when(wid < NT)
    def _():
        pltpu.sync_copy(ones_v, acc.at[_ck(dst_v, NCW)], add=True)

    plsc.subcore_barrier()
    pltpu.sync_copy(acc.at[pl.ds(sid * RPT, RPT)],
                    outP.at[pl.ds(cid * NP + sid * RPT, RPT)])


def _ck(idx_v, j):
    return idx_v.at[pl.ds(j * CH, CH)]


def _prop_loop(SRC, src_v, dst_v, rows_v, acc, sem_g, sem_s, wid):
    for d in range(D):
        pltpu.async_copy(SRC.at[_ck(src_v, d)], rows_v.at[d], sem_g)

    def group(gi, c):
        for b in range(NBUF):
            j = gi * NBUF + b
            pltpu.make_async_copy(SRC.at[_ck(src_v, j)], rows_v.at[b],
                                  sem_g).wait()
            pltpu.async_copy(rows_v.at[b], acc.at[_ck(dst_v, j)], sem_s,
                             add=True)

            @pl.when(j >= NBUF - D)
            def _():
                pltpu.make_async_copy(rows_v.at[b], acc.at[_ck(dst_v, b)],
                                      sem_s).wait()

            @pl.when(j + D < NCW)
            def _():
                pltpu.async_copy(SRC.at[_ck(src_v, j + D)],
                                 rows_v.at[(b + D) % NBUF], sem_g)

        return c

    lax.fori_loop(0, NGRP, group, 0)

    @pl.when(wid < NT)
    def _():
        pltpu.async_copy(SRC.at[_ck(src_v, NCW)], rows_v.at[0], sem_g)
        pltpu.make_async_copy(SRC.at[_ck(src_v, NCW)], rows_v.at[0],
                              sem_g).wait()
        pltpu.async_copy(rows_v.at[0], acc.at[_ck(dst_v, NCW)], sem_s,
                         add=True)
        pltpu.make_async_copy(rows_v.at[0], acc.at[_ck(dst_v, NCW)],
                              sem_s).wait()

    for k in range(NBUF - D):
        pltpu.make_async_copy(rows_v.at[k], acc.at[_ck(dst_v, k)], sem_s).wait()


def _prop1_body(g, EI, zeros16, outP, src_v, dst_v, rows_v, g0s, acc,
                sem_g, sem_s):
    cid, sid, wid = _worker_ids()
    _stage_chunks(EI, 0, src_v, wid)
    _stage_chunks(EI, 1, dst_v, wid)
    pltpu.sync_copy(zeros16.at[pl.ds(sid * RPT, RPT)],
                    acc.at[pl.ds(sid * RPT, RPT)])
    pltpu.sync_copy(g.at[pl.ds(sid * RPT, RPT)], g0s.at[pl.ds(sid * RPT, RPT)])
    plsc.subcore_barrier()
    _prop_loop(g0s, src_v, dst_v, rows_v, acc, sem_g, sem_s, wid)
    plsc.subcore_barrier()
    pltpu.sync_copy(acc.at[pl.ds(sid * RPT, RPT)],
                    outP.at[pl.ds(cid * NP + sid * RPT, RPT)])


def _prop2g_body(pP, EI, niI, dinv, zeros16, G, Dni,
                 src_v, dst_v, rows_v, pv0, pv1, dv, gv, ni_v, grows_v,
                 g1s, acc, sem_g, sem_s):
    cid, sid, wid = _worker_ids()
    _stage_chunks(EI, 0, src_v, wid)
    _stage_chunks(EI, 1, dst_v, wid)
    pltpu.sync_copy(zeros16.at[pl.ds(sid * RPT, RPT)],
                    acc.at[pl.ds(sid * RPT, RPT)])

    r0 = sid * RPT
    pltpu.sync_copy(pP.at[pl.ds(r0, RPT)], pv0)
    pltpu.sync_copy(pP.at[pl.ds(NP + r0, RPT)], pv1)
    pltpu.sync_copy(dinv.at[pl.ds(r0, RPT)], dv)

    def row(i, c):
        t = (pv0[i] + pv1[i]) * dv[i]
        gv[i] = jnp.maximum(t, 0.0) * dv[i]
        return c

    lax.fori_loop(0, RPT, row, 0)
    pltpu.sync_copy(gv, g1s.at[pl.ds(r0, RPT)])
    plsc.subcore_barrier()

    _prop_loop(g1s, src_v, dst_v, rows_v, acc, sem_g, sem_s, wid)
    plsc.subcore_barrier()

    @pl.when(sid < 8)
    def _():
        pltpu.sync_copy(niI.at[pl.ds(sid * CH, CH)], ni_v)
        pltpu.async_copy(acc.at[ni_v], grows_v, sem_g).wait()
        pltpu.sync_copy(grows_v, G.at[pl.ds(cid * NI_P + sid * CH, CH)])

    @pl.when((sid >= 8) & (cid == 0))
    def _():
        pltpu.sync_copy(niI.at[pl.ds((sid - 8) * CH, CH)], ni_v)
        pltpu.async_copy(dinv.at[ni_v], grows_v, sem_g).wait()
        pltpu.sync_copy(grows_v, Dni.at[pl.ds((sid - 8) * CH, CH)])


@functools.lru_cache(maxsize=None)
def _sc_kernels():
    mesh = plsc.VectorSubcoreMesh(core_axis_name="c", subcore_axis_name="s",
                                  num_cores=NC, num_subcores=NS)
    params = pltpu.CompilerParams(use_tc_tiling_on_sc=False)
    deg = pl.kernel(
        _deg_body,
        out_type=jax.ShapeDtypeStruct((NC * NP,), jnp.float32),
        mesh=mesh,
        compiler_params=params,
        scratch_types=[
            pltpu.VMEM(((NCW + 1) * CH,), jnp.int32),
            pltpu.VMEM((CH,), jnp.float32),
            pltpu.VMEM_SHARED((NP,), jnp.float32),
            pltpu.SemaphoreType.DMA,
        ],
    )
    prop1 = pl.kernel(
        _prop1_body,
        out_type=jax.ShapeDtypeStruct((NC * NP, H), jnp.float32),
        mesh=mesh,
        compiler_params=params,
        scratch_types=[
            pltpu.VMEM(((NCW + 1) * CH,), jnp.int32),
            pltpu.VMEM(((NCW + 1) * CH,), jnp.int32),
            pltpu.VMEM((NBUF, CH, H), jnp.float32),
            pltpu.VMEM_SHARED((NP, H), jnp.float32),
            pltpu.VMEM_SHARED((NP, H), jnp.float32),
            pltpu.SemaphoreType.DMA,
            pltpu.SemaphoreType.DMA,
        ],
    )
    prop2g = pl.kernel(
        _prop2g_body,
        out_type=(jax.ShapeDtypeStruct((NC * NI_P, H), jnp.float32),
                  jax.ShapeDtypeStruct((NI_P, H), jnp.float32)),
        mesh=mesh,
        compiler_params=params,
        scratch_types=[
            pltpu.VMEM(((NCW + 1) * CH,), jnp.int32),
            pltpu.VMEM(((NCW + 1) * CH,), jnp.int32),
            pltpu.VMEM((NBUF, CH, H), jnp.float32),
            pltpu.VMEM((RPT, H), jnp.float32),
            pltpu.VMEM((RPT, H), jnp.float32),
            pltpu.VMEM((RPT, H), jnp.float32),
            pltpu.VMEM((RPT, H), jnp.float32),
            pltpu.VMEM((CH,), jnp.int32),
            pltpu.VMEM((CH, H), jnp.float32),
            pltpu.VMEM_SHARED((NP, H), jnp.float32),
            pltpu.VMEM_SHARED((NP, H), jnp.float32),
            pltpu.SemaphoreType.DMA,
            pltpu.SemaphoreType.DMA,
        ],
    )
    return deg, prop1, prop2g


def _fuse1_body(x_ref, w1_ref, degp_ref, dinv_ref, g0_ref):
    deg = jnp.maximum(degp_ref[:NP] + degp_ref[NP:], 1.0)
    dinv = lax.rsqrt(deg)
    dinv16 = jnp.broadcast_to(dinv[:, None], (NP, H))
    dinv_ref[...] = dinv16
    xw = jnp.dot(x_ref[...], w1_ref[...], preferred_element_type=jnp.float32)
    xwp = jnp.concatenate([xw, jnp.zeros((NP - N, H), jnp.float32)], axis=0)
    g0_ref[...] = xwp * dinv16


_fuse1 = pl.pallas_call(
    _fuse1_body,
    out_shape=(jax.ShapeDtypeStruct((NP, H), jnp.float32),
               jax.ShapeDtypeStruct((NP, H), jnp.float32)),
)


def _fuse3_body(G_ref, dni_ref, w2_ref, out_ref):
    rows = (G_ref[:NI_P] + G_ref[NI_P:]) * dni_ref[...]
    out_ref[...] = jnp.dot(rows, w2_ref[...],
                           preferred_element_type=jnp.float32)


_fuse3 = pl.pallas_call(
    _fuse3_body,
    out_shape=jax.ShapeDtypeStruct((NI_P, OUT), jnp.float32),
)


def kernel(x, edge_index, node_index, W1, W2):
    deg_k, prop1_k, prop2g_k = _sc_kernels()
    EI = edge_index.astype(jnp.int32)
    niP = jnp.concatenate([node_index.astype(jnp.int32),
                           jnp.zeros((NI_P - NI,), jnp.int32)])
    zeros1 = jnp.zeros((NP,), jnp.float32)
    zeros16 = jnp.zeros((NP, H), jnp.float32)
    ones1 = jnp.ones((CH,), jnp.float32)

    degp = deg_k(EI, ones1, zeros1)
    dinv16, g0 = _fuse1(x, W1, degp)
    pP = prop1_k(g0, EI, zeros16)
    G, Dni = prop2g_k(pP, EI, niP, dinv16, zeros16)
    out = _fuse3(G, Dni, W2)
    return out[:NI]
